# Initial kernel scaffold; baseline (speedup 1.0000x reference)
import jax
import jax.numpy as jnp
from jax.experimental import pallas as pl


def kernel(x, angle, tx, ty):
    raise NotImplementedError("write your pallas kernel here")


# speedup vs baseline: 1.4479x; 1.4479x over previous
import math

import numpy as np
import jax
import jax.numpy as jnp
from jax.experimental import pallas as pl
from jax.experimental.pallas import tpu as pltpu

_H = _W = 32
_HW = _H * _W


def _filter_operator_T_np() -> np.ndarray:
    x = np.linspace(-1.0, 1.0, _H, dtype=np.float32)
    xx, yy = np.meshgrid(x, x, indexing="ij")
    radius = np.sqrt(xx ** 2 + yy ** 2)
    ratio, fwhm = 0.5, 1.0
    r = radius / ratio
    r_safe = np.where(radius == 0, 1.0, r)
    sinc = np.where(radius == 0, 1.0, np.sin(r_safe) / r_safe)
    gauss = np.exp(-0.5 * (radius / fwhm) ** 2)
    filt = (sinc * gauss).astype(np.float32)
    k_shifted = np.fft.ifftshift(filt)
    basis = np.eye(_HW, dtype=np.float32).reshape(_HW, _H, _W)
    resp = np.fft.ifft2(np.fft.fft2(basis) * k_shifted).real
    return resp.reshape(_HW, _HW).astype(np.float32)


_LT_F32 = _filter_operator_T_np()
try:
    import ml_dtypes

    _LT_BF16 = _LT_F32.astype(ml_dtypes.bfloat16)
except ImportError:
    _LT_BF16 = None


def _build_combined_kernel(s_ref, lt_ref, a_ref):
    cos_a = s_ref[0, 0]
    sin_a = s_ref[0, 1]
    tx = s_ref[0, 2]
    ty = s_ref[0, 3]

    o = jax.lax.broadcasted_iota(jnp.int32, (1, _HW), 1)
    wf = (o & (_W - 1)).astype(jnp.float32)
    hf = (o >> 5).astype(jnp.float32)
    xg = (2.0 * wf + 1.0) / _W - 1.0
    yg = (2.0 * hf + 1.0) / _H - 1.0
    gx = cos_a * xg - sin_a * yg + tx
    gy = sin_a * xg + cos_a * yg + ty
    ix = ((gx + 1.0) * _W - 1.0) / 2.0
    iy = ((gy + 1.0) * _H - 1.0) / 2.0

    x0 = jnp.floor(ix)
    y0 = jnp.floor(iy)
    x1 = x0 + 1.0
    y1 = y0 + 1.0
    wx1 = ix - x0
    wx0 = 1.0 - wx1
    wy1 = iy - y0
    wy0 = 1.0 - wy1

    blk = a_ref.shape[0]
    rows = (jax.lax.broadcasted_iota(jnp.int32, (blk, _HW), 0)
            + pl.program_id(0) * blk)

    m = jnp.zeros((blk, _HW), jnp.float32)
    for yc, xc, wgt in ((y0, x0, wy0 * wx0), (y0, x1, wy0 * wx1),
                        (y1, x0, wy1 * wx0), (y1, x1, wy1 * wx1)):
        valid = ((yc >= 0.0) & (yc < float(_H))
                 & (xc >= 0.0) & (xc < float(_W)))
        w_ok = jnp.where(valid, wgt, 0.0)
        idx = (yc * float(_W) + xc).astype(jnp.int32)
        m = m + jnp.where(rows == idx, w_ok, 0.0)

    a_ref[...] = jnp.dot(m.astype(jnp.bfloat16), lt_ref[...],
                         preferred_element_type=jnp.float32
                         ).astype(jnp.bfloat16)


def _build_combined(cos_a, sin_a, tx, ty):
    s = jnp.stack([cos_a, sin_a, tx, ty]).reshape(1, 4).astype(jnp.float32)
    if _LT_BF16 is not None:
        lt = jnp.asarray(_LT_BF16)
    else:
        lt = jnp.asarray(_LT_F32).astype(jnp.bfloat16)
    blk = _HW // 2
    return pl.pallas_call(
        _build_combined_kernel,
        grid=(2,),
        in_specs=[pl.BlockSpec(memory_space=pltpu.SMEM),
                  pl.BlockSpec((_HW, _HW), lambda i: (0, 0))],
        out_specs=pl.BlockSpec((blk, _HW), lambda i: (i, 0)),
        out_shape=jax.ShapeDtypeStruct((_HW, _HW), jnp.bfloat16),
        compiler_params=pltpu.CompilerParams(
            dimension_semantics=("arbitrary",)),
    )(s, lt)


def _apply_kernel(x_ref, a_ref, o_ref):
    o_ref[...] = jnp.dot(x_ref[...].astype(jnp.bfloat16), a_ref[...],
                         preferred_element_type=jnp.float32)


def kernel(x, angle, tx, ty):
    C, D, H, W = x.shape
    assert H == _H and W == _W, "operator table is built for 32x32 slices"
    B = C * D
    x_flat = x.reshape(B, _HW).astype(jnp.float32)

    a = angle * (math.pi / 180.0)
    A = _build_combined(jnp.cos(a), jnp.sin(a), tx, ty)

    tb = next((c for c in (1024, 512, 256, 128, 64, 32, 16, 8)
               if B % c == 0), None)
    if tb is None:
        y_flat = pl.pallas_call(
            _apply_kernel,
            out_shape=jax.ShapeDtypeStruct((B, _HW), jnp.float32),
        )(x_flat, A)
    else:
        y_flat = pl.pallas_call(
            _apply_kernel,
            grid=(B // tb,),
            in_specs=[pl.BlockSpec((tb, _HW), lambda i: (i, 0)),
                      pl.BlockSpec((_HW, _HW), lambda i: (0, 0))],
            out_specs=pl.BlockSpec((tb, _HW), lambda i: (i, 0)),
            out_shape=jax.ShapeDtypeStruct((B, _HW), jnp.float32),
            compiler_params=pltpu.CompilerParams(
                dimension_semantics=("arbitrary",)),
        )(x_flat, A)
    return y_flat.reshape(C, D, H, W)


# speedup vs baseline: 8.4208x; 5.8161x over previous
import math

import numpy as np
import jax
import jax.numpy as jnp
from jax.experimental import pallas as pl
from jax.experimental.pallas import tpu as pltpu

_H = _W = 32
_HW = _H * _W


def _filter_operator_np() -> np.ndarray:
    x = np.linspace(-1.0, 1.0, _H, dtype=np.float32)
    xx, yy = np.meshgrid(x, x, indexing="ij")
    radius = np.sqrt(xx ** 2 + yy ** 2)
    ratio, fwhm = 0.5, 1.0
    r = radius / ratio
    r_safe = np.where(radius == 0, 1.0, r)
    sinc = np.where(radius == 0, 1.0, np.sin(r_safe) / r_safe)
    gauss = np.exp(-0.5 * (radius / fwhm) ** 2)
    filt = (sinc * gauss).astype(np.float32)
    k_shifted = np.fft.ifftshift(filt)
    basis = np.eye(_HW, dtype=np.float32).reshape(_HW, _H, _W)
    resp = np.fft.ifft2(np.fft.fft2(basis) * k_shifted).real
    return np.ascontiguousarray(resp.reshape(_HW, _HW).T.astype(np.float32))


_L_F32 = _filter_operator_np()
try:
    import ml_dtypes

    _L_BF16 = _L_F32.astype(ml_dtypes.bfloat16)
except ImportError:
    _L_BF16 = None


def _build_combined_kernel(s_ref, l_ref, at_ref):
    cos_a = s_ref[0, 0]
    sin_a = s_ref[0, 1]
    tx = s_ref[0, 2]
    ty = s_ref[0, 3]

    o = jax.lax.broadcasted_iota(jnp.int32, (1, _HW), 1)
    wf = (o & (_W - 1)).astype(jnp.float32)
    hf = (o >> 5).astype(jnp.float32)
    xg = (2.0 * wf + 1.0) / _W - 1.0
    yg = (2.0 * hf + 1.0) / _H - 1.0
    gx = cos_a * xg - sin_a * yg + tx
    gy = sin_a * xg + cos_a * yg + ty
    ix = ((gx + 1.0) * _W - 1.0) / 2.0
    iy = ((gy + 1.0) * _H - 1.0) / 2.0

    x0 = jnp.floor(ix)
    y0 = jnp.floor(iy)
    x1 = x0 + 1.0
    y1 = y0 + 1.0
    wx1 = ix - x0
    wx0 = 1.0 - wx1
    wy1 = iy - y0
    wy0 = 1.0 - wy1

    rows = jax.lax.broadcasted_iota(jnp.int32, (_HW, _HW), 0)

    m = jnp.zeros((_HW, _HW), jnp.float32)
    for yc, xc, wgt in ((y0, x0, wy0 * wx0), (y0, x1, wy0 * wx1),
                        (y1, x0, wy1 * wx0), (y1, x1, wy1 * wx1)):
        valid = ((yc >= 0.0) & (yc < float(_H))
                 & (xc >= 0.0) & (xc < float(_W)))
        w_ok = jnp.where(valid, wgt, 0.0)
        idx = (yc * float(_W) + xc).astype(jnp.int32)
        m = m + jnp.where(rows == idx, w_ok, 0.0)

    at_ref[...] = jax.lax.dot_general(
        l_ref[...], m.astype(jnp.bfloat16),
        (((1,), (1,)), ((), ())),
        preferred_element_type=jnp.float32).astype(jnp.bfloat16)


def _build_combined_T(cos_a, sin_a, tx, ty):
    s = jnp.stack([cos_a, sin_a, tx, ty]).reshape(1, 4).astype(jnp.float32)
    if _L_BF16 is not None:
        lmat = jnp.asarray(_L_BF16)
    else:
        lmat = jnp.asarray(_L_F32).astype(jnp.bfloat16)
    return pl.pallas_call(
        _build_combined_kernel,
        in_specs=[pl.BlockSpec(memory_space=pltpu.SMEM),
                  pl.BlockSpec((_HW, _HW), lambda: (0, 0))],
        out_specs=pl.BlockSpec((_HW, _HW), lambda: (0, 0)),
        out_shape=jax.ShapeDtypeStruct((_HW, _HW), jnp.bfloat16),
    )(s, lmat)


def _apply_kernel(at_ref, x_ref, o_ref):
    o_ref[...] = jnp.dot(at_ref[...], x_ref[...].astype(jnp.bfloat16),
                         preferred_element_type=jnp.float32)


def kernel(x, angle, tx, ty):
    C, D, H, W = x.shape
    assert H == _H and W == _W, "operator table is built for 32x32 slices"

    a = angle * (math.pi / 180.0)
    at = _build_combined_T(jnp.cos(a), jnp.sin(a), tx, ty)

    x2 = jnp.transpose(x, (0, 2, 3, 1)).reshape(C * _HW, D)

    y2 = pl.pallas_call(
        _apply_kernel,
        grid=(C,),
        in_specs=[pl.BlockSpec((_HW, _HW), lambda i: (0, 0)),
                  pl.BlockSpec((_HW, D), lambda i: (i, 0))],
        out_specs=pl.BlockSpec((_HW, D), lambda i: (i, 0)),
        out_shape=jax.ShapeDtypeStruct((C * _HW, D), jnp.float32),
        compiler_params=pltpu.CompilerParams(
            dimension_semantics=("arbitrary",)),
    )(at, x2)

    return jnp.transpose(y2.reshape(C, _H, _W, D), (0, 3, 1, 2))


# speedup vs baseline: 9.4839x; 1.1263x over previous
import math

import numpy as np
import jax
import jax.numpy as jnp
from jax.experimental import pallas as pl
from jax.experimental.pallas import tpu as pltpu

_H = _W = 32
_HW = _H * _W


def _filter_operator_np() -> np.ndarray:
    x = np.linspace(-1.0, 1.0, _H, dtype=np.float32)
    xx, yy = np.meshgrid(x, x, indexing="ij")
    radius = np.sqrt(xx ** 2 + yy ** 2)
    ratio, fwhm = 0.5, 1.0
    r = radius / ratio
    r_safe = np.where(radius == 0, 1.0, r)
    sinc = np.where(radius == 0, 1.0, np.sin(r_safe) / r_safe)
    gauss = np.exp(-0.5 * (radius / fwhm) ** 2)
    filt = (sinc * gauss).astype(np.float32)
    k_shifted = np.fft.ifftshift(filt)
    basis = np.eye(_HW, dtype=np.float32).reshape(_HW, _H, _W)
    resp = np.fft.ifft2(np.fft.fft2(basis) * k_shifted).real
    return np.ascontiguousarray(resp.reshape(_HW, _HW).T.astype(np.float32))


_L_F32 = _filter_operator_np()
try:
    import ml_dtypes

    _L_BF16 = _L_F32.astype(ml_dtypes.bfloat16)
except ImportError:
    _L_BF16 = None


def _build_operator(s_ref, l_ref, at_ref):
    cos_a = s_ref[0, 0]
    sin_a = s_ref[0, 1]
    tx = s_ref[0, 2]
    ty = s_ref[0, 3]

    o = jax.lax.broadcasted_iota(jnp.int32, (1, _HW), 1)
    wf = (o & (_W - 1)).astype(jnp.float32)
    hf = (o >> 5).astype(jnp.float32)
    xg = (2.0 * wf + 1.0) / _W - 1.0
    yg = (2.0 * hf + 1.0) / _H - 1.0
    gx = cos_a * xg - sin_a * yg + tx
    gy = sin_a * xg + cos_a * yg + ty
    ix = ((gx + 1.0) * _W - 1.0) / 2.0
    iy = ((gy + 1.0) * _H - 1.0) / 2.0

    x0 = jnp.floor(ix)
    y0 = jnp.floor(iy)
    x1 = x0 + 1.0
    y1 = y0 + 1.0
    wx1 = ix - x0
    wx0 = 1.0 - wx1
    wy1 = iy - y0
    wy0 = 1.0 - wy1

    rows = jax.lax.broadcasted_iota(jnp.int32, (_HW, _HW), 0)

    m = jnp.zeros((_HW, _HW), jnp.float32)
    for yc, xc, wgt in ((y0, x0, wy0 * wx0), (y0, x1, wy0 * wx1),
                        (y1, x0, wy1 * wx0), (y1, x1, wy1 * wx1)):
        valid = ((yc >= 0.0) & (yc < float(_H))
                 & (xc >= 0.0) & (xc < float(_W)))
        w_ok = jnp.where(valid, wgt, 0.0)
        idx = (yc * float(_W) + xc).astype(jnp.int32)
        m = m + jnp.where(rows == idx, w_ok, 0.0)

    at_ref[...] = jax.lax.dot_general(
        l_ref[...], m.astype(jnp.bfloat16),
        (((1,), (1,)), ((), ())),
        preferred_element_type=jnp.float32).astype(jnp.bfloat16)


def _fused_kernel(s_ref, l_ref, x_ref, o_ref, at_ref):
    @pl.when(pl.program_id(0) == 0)
    def _():
        _build_operator(s_ref, l_ref, at_ref)

    at = at_ref[...]
    nslc = x_ref.shape[0] // _HW
    for k in range(nslc):
        sl = slice(k * _HW, (k + 1) * _HW)
        o_ref[sl, :] = jnp.dot(at, x_ref[sl, :].astype(jnp.bfloat16),
                               preferred_element_type=jnp.float32)


def kernel(x, angle, tx, ty):
    C, D, H, W = x.shape
    assert H == _H and W == _W, "operator table is built for 32x32 slices"

    a = angle * (math.pi / 180.0)
    s = jnp.stack([jnp.cos(a), jnp.sin(a), tx, ty]).reshape(1, 4)
    s = s.astype(jnp.float32)
    if _L_BF16 is not None:
        lmat = jnp.asarray(_L_BF16)
    else:
        lmat = jnp.asarray(_L_F32).astype(jnp.bfloat16)

    x2 = jnp.transpose(x, (0, 2, 3, 1)).reshape(C * _HW, D)

    slc = 2 if C % 2 == 0 else 1
    blk = slc * _HW
    y2 = pl.pallas_call(
        _fused_kernel,
        grid=(C // slc,),
        in_specs=[pl.BlockSpec(memory_space=pltpu.SMEM),
                  pl.BlockSpec((_HW, _HW), lambda i: (0, 0)),
                  pl.BlockSpec((blk, D), lambda i: (i, 0))],
        out_specs=pl.BlockSpec((blk, D), lambda i: (i, 0)),
        out_shape=jax.ShapeDtypeStruct((C * _HW, D), jnp.float32),
        scratch_shapes=[pltpu.VMEM((_HW, _HW), jnp.bfloat16)],
        compiler_params=pltpu.CompilerParams(
            dimension_semantics=("arbitrary",),
            vmem_limit_bytes=52 * 1024 * 1024),
    )(s, lmat, x2)

    return jnp.transpose(y2.reshape(C, _H, _W, D), (0, 3, 1, 2))


# speedup vs baseline: 9.9511x; 1.0493x over previous
import math

import numpy as np
import jax
import jax.numpy as jnp
from jax.experimental import pallas as pl
from jax.experimental.pallas import tpu as pltpu

_H = _W = 32
_HW = _H * _W


def _filter_operator_np() -> np.ndarray:
    x = np.linspace(-1.0, 1.0, _H, dtype=np.float32)
    xx, yy = np.meshgrid(x, x, indexing="ij")
    radius = np.sqrt(xx ** 2 + yy ** 2)
    ratio, fwhm = 0.5, 1.0
    r = radius / ratio
    r_safe = np.where(radius == 0, 1.0, r)
    sinc = np.where(radius == 0, 1.0, np.sin(r_safe) / r_safe)
    gauss = np.exp(-0.5 * (radius / fwhm) ** 2)
    filt = (sinc * gauss).astype(np.float32)
    k_shifted = np.fft.ifftshift(filt)
    basis = np.eye(_HW, dtype=np.float32).reshape(_HW, _H, _W)
    resp = np.fft.ifft2(np.fft.fft2(basis) * k_shifted).real
    return np.ascontiguousarray(resp.reshape(_HW, _HW).T.astype(np.float32))


_L_F32 = _filter_operator_np()
try:
    import ml_dtypes

    _L_BF16 = _L_F32.astype(ml_dtypes.bfloat16)
except ImportError:
    _L_BF16 = None


def _build_operator(s_ref, l_ref, at_ref):
    cos_a = s_ref[0, 0]
    sin_a = s_ref[0, 1]
    tx = s_ref[0, 2]
    ty = s_ref[0, 3]

    o = jax.lax.broadcasted_iota(jnp.int32, (1, _HW), 1)
    wf = (o & (_W - 1)).astype(jnp.float32)
    hf = (o >> 5).astype(jnp.float32)
    xg = (2.0 * wf + 1.0) / _W - 1.0
    yg = (2.0 * hf + 1.0) / _H - 1.0
    gx = cos_a * xg - sin_a * yg + tx
    gy = sin_a * xg + cos_a * yg + ty
    ix = ((gx + 1.0) * _W - 1.0) / 2.0
    iy = ((gy + 1.0) * _H - 1.0) / 2.0

    x0 = jnp.floor(ix)
    y0 = jnp.floor(iy)
    x1 = x0 + 1.0
    y1 = y0 + 1.0
    wx1 = ix - x0
    wx0 = 1.0 - wx1
    wy1 = iy - y0
    wy0 = 1.0 - wy1

    rows = jax.lax.broadcasted_iota(jnp.int32, (_HW, _HW), 0)

    m = jnp.zeros((_HW, _HW), jnp.float32)
    for yc, xc, wgt in ((y0, x0, wy0 * wx0), (y0, x1, wy0 * wx1),
                        (y1, x0, wy1 * wx0), (y1, x1, wy1 * wx1)):
        valid = ((yc >= 0.0) & (yc < float(_H))
                 & (xc >= 0.0) & (xc < float(_W)))
        w_ok = jnp.where(valid, wgt, 0.0)
        idx = (yc * float(_W) + xc).astype(jnp.int32)
        m = m + jnp.where(rows == idx, w_ok, 0.0)

    at_ref[...] = jax.lax.dot_general(
        l_ref[...], m.astype(jnp.bfloat16),
        (((1,), (1,)), ((), ())),
        preferred_element_type=jnp.float32).astype(jnp.bfloat16)


def _fused_kernel(nsteps, blk, s_ref, l_ref, x_ref, o_ref,
                  at_ref, xbuf_ref, sem_ref):
    i = pl.program_id(0)
    slot = jax.lax.rem(i, 2)
    nxt = jax.lax.rem(i + 1, 2)

    @pl.when(i == 0)
    def _():
        pltpu.make_async_copy(x_ref.at[pl.ds(0, blk), :], xbuf_ref.at[0],
                              sem_ref.at[0]).start()
        _build_operator(s_ref, l_ref, at_ref)

    @pl.when(i + 1 < nsteps)
    def _():
        pltpu.make_async_copy(x_ref.at[pl.ds((i + 1) * blk, blk), :],
                              xbuf_ref.at[nxt], sem_ref.at[nxt]).start()

    pltpu.make_async_copy(xbuf_ref.at[slot], xbuf_ref.at[slot],
                          sem_ref.at[slot]).wait()

    at = at_ref[...]
    for k in range(blk // _HW):
        sl = slice(k * _HW, (k + 1) * _HW)
        o_ref[sl, :] = jnp.dot(at, xbuf_ref[slot, sl, :].astype(jnp.bfloat16),
                               preferred_element_type=jnp.float32)


def kernel(x, angle, tx, ty):
    C, D, H, W = x.shape
    assert H == _H and W == _W, "operator table is built for 32x32 slices"

    a = angle * (math.pi / 180.0)
    s = jnp.stack([jnp.cos(a), jnp.sin(a), tx, ty]).reshape(1, 4)
    s = s.astype(jnp.float32)
    if _L_BF16 is not None:
        lmat = jnp.asarray(_L_BF16)
    else:
        lmat = jnp.asarray(_L_F32).astype(jnp.bfloat16)

    x2 = jnp.transpose(x, (0, 2, 3, 1)).reshape(C * _HW, D)

    slc = 2 if C % 2 == 0 else 1
    blk = slc * _HW
    nsteps = C // slc
    import functools
    body = functools.partial(_fused_kernel, nsteps, blk)
    y2 = pl.pallas_call(
        body,
        grid=(nsteps,),
        in_specs=[pl.BlockSpec(memory_space=pltpu.SMEM),
                  pl.BlockSpec((_HW, _HW), lambda i: (0, 0)),
                  pl.BlockSpec(memory_space=pl.ANY)],
        out_specs=pl.BlockSpec((blk, D), lambda i: (i, 0)),
        out_shape=jax.ShapeDtypeStruct((C * _HW, D), jnp.float32),
        scratch_shapes=[pltpu.VMEM((_HW, _HW), jnp.bfloat16),
                        pltpu.VMEM((2, blk, D), jnp.float32),
                        pltpu.SemaphoreType.DMA((2,))],
        compiler_params=pltpu.CompilerParams(
            dimension_semantics=("arbitrary",),
            vmem_limit_bytes=52 * 1024 * 1024),
    )(s, lmat, x2)

    return jnp.transpose(y2.reshape(C, _H, _W, D), (0, 3, 1, 2))


# speedup vs baseline: 10.8296x; 1.0883x over previous
import math

import numpy as np
import jax
import jax.numpy as jnp
from jax.experimental import pallas as pl
from jax.experimental.pallas import tpu as pltpu

_H = _W = 32
_HW = _H * _W


def _filter_operator_np() -> np.ndarray:
    x = np.linspace(-1.0, 1.0, _H, dtype=np.float32)
    xx, yy = np.meshgrid(x, x, indexing="ij")
    radius = np.sqrt(xx ** 2 + yy ** 2)
    ratio, fwhm = 0.5, 1.0
    r = radius / ratio
    r_safe = np.where(radius == 0, 1.0, r)
    sinc = np.where(radius == 0, 1.0, np.sin(r_safe) / r_safe)
    gauss = np.exp(-0.5 * (radius / fwhm) ** 2)
    filt = (sinc * gauss).astype(np.float32)
    k_shifted = np.fft.ifftshift(filt)
    basis = np.eye(_HW, dtype=np.float32).reshape(_HW, _H, _W)
    resp = np.fft.ifft2(np.fft.fft2(basis) * k_shifted).real
    return np.ascontiguousarray(resp.reshape(_HW, _HW).T.astype(np.float32))


_L_F32 = _filter_operator_np()
try:
    import ml_dtypes

    _L_BF16 = _L_F32.astype(ml_dtypes.bfloat16)
except ImportError:
    _L_BF16 = None


def _build_operator(s_ref, l_ref, at_ref):
    cos_a = s_ref[0, 0]
    sin_a = s_ref[0, 1]
    tx = s_ref[0, 2]
    ty = s_ref[0, 3]

    o = jax.lax.broadcasted_iota(jnp.int32, (1, _HW), 1)
    wf = (o & (_W - 1)).astype(jnp.float32)
    hf = (o >> 5).astype(jnp.float32)
    xg = (2.0 * wf + 1.0) / _W - 1.0
    yg = (2.0 * hf + 1.0) / _H - 1.0
    gx = cos_a * xg - sin_a * yg + tx
    gy = sin_a * xg + cos_a * yg + ty
    ix = ((gx + 1.0) * _W - 1.0) / 2.0
    iy = ((gy + 1.0) * _H - 1.0) / 2.0

    x0 = jnp.floor(ix)
    y0 = jnp.floor(iy)
    x1 = x0 + 1.0
    y1 = y0 + 1.0
    wx1 = ix - x0
    wx0 = 1.0 - wx1
    wy1 = iy - y0
    wy0 = 1.0 - wy1

    rows = jax.lax.broadcasted_iota(jnp.int32, (_HW, _HW), 0)

    m = jnp.zeros((_HW, _HW), jnp.float32)
    for yc, xc, wgt in ((y0, x0, wy0 * wx0), (y0, x1, wy0 * wx1),
                        (y1, x0, wy1 * wx0), (y1, x1, wy1 * wx1)):
        valid = ((yc >= 0.0) & (yc < float(_H))
                 & (xc >= 0.0) & (xc < float(_W)))
        w_ok = jnp.where(valid, wgt, 0.0)
        idx = (yc * float(_W) + xc).astype(jnp.int32)
        m = m + jnp.where(rows == idx, w_ok, 0.0)

    at_ref[...] = jax.lax.dot_general(
        l_ref[...], m.astype(jnp.bfloat16),
        (((1,), (1,)), ((), ())),
        preferred_element_type=jnp.float32).astype(jnp.bfloat16)


def _fused_kernel(nsteps, blk, s_ref, l_ref, x_ref, o_ref,
                  at_ref, xbuf_ref, obuf_ref, isem_ref, osem_ref):
    i = pl.program_id(0)
    slot = jax.lax.rem(i, 2)
    nxt = jax.lax.rem(i + 1, 2)
    nh = blk // _HW

    def start_in(step, sl):
        base = step * blk
        for h in range(nh):
            pltpu.make_async_copy(
                x_ref.at[pl.ds(base + h * _HW, _HW), :],
                xbuf_ref.at[sl, pl.ds(h * _HW, _HW), :],
                isem_ref.at[sl, h]).start()

    @pl.when(i == 0)
    def _():
        start_in(0, slot)
        _build_operator(s_ref, l_ref, at_ref)

    @pl.when(i + 1 < nsteps)
    def _():
        start_in(i + 1, nxt)

    at = at_ref[...]
    for k in range(nh):
        sl = pl.ds(k * _HW, _HW)
        pltpu.make_async_copy(xbuf_ref.at[slot, sl, :],
                              xbuf_ref.at[slot, sl, :],
                              isem_ref.at[slot, k]).wait()

        @pl.when(i >= 2)
        def _():
            pltpu.make_async_copy(obuf_ref.at[slot, sl, :],
                                  obuf_ref.at[slot, sl, :],
                                  osem_ref.at[slot, k]).wait()

        obuf_ref[slot, sl, :] = jnp.dot(
            at, xbuf_ref[slot, sl, :].astype(jnp.bfloat16),
            preferred_element_type=jnp.float32)

        pltpu.make_async_copy(obuf_ref.at[slot, sl, :],
                              o_ref.at[pl.ds(i * blk + k * _HW, _HW), :],
                              osem_ref.at[slot, k]).start()

    @pl.when(i == nsteps - 1)
    def _():
        for k in range(nh):
            sl = pl.ds(k * _HW, _HW)
            if nsteps >= 2:
                pltpu.make_async_copy(obuf_ref.at[nxt, sl, :],
                                      obuf_ref.at[nxt, sl, :],
                                      osem_ref.at[nxt, k]).wait()
            pltpu.make_async_copy(obuf_ref.at[slot, sl, :],
                                  obuf_ref.at[slot, sl, :],
                                  osem_ref.at[slot, k]).wait()


def kernel(x, angle, tx, ty):
    C, D, H, W = x.shape
    assert H == _H and W == _W, "operator table is built for 32x32 slices"

    a = angle * (math.pi / 180.0)
    s = jnp.stack([jnp.cos(a), jnp.sin(a), tx, ty]).reshape(1, 4)
    s = s.astype(jnp.float32)
    if _L_BF16 is not None:
        lmat = jnp.asarray(_L_BF16)
    else:
        lmat = jnp.asarray(_L_F32).astype(jnp.bfloat16)

    x2 = jnp.transpose(x, (0, 2, 3, 1)).reshape(C * _HW, D)

    slc = 2 if C % 2 == 0 else 1
    blk = slc * _HW
    nsteps = C // slc
    import functools
    body = functools.partial(_fused_kernel, nsteps, blk)
    y2 = pl.pallas_call(
        body,
        grid=(nsteps,),
        in_specs=[pl.BlockSpec(memory_space=pltpu.SMEM),
                  pl.BlockSpec((_HW, _HW), lambda i: (0, 0)),
                  pl.BlockSpec(memory_space=pl.ANY)],
        out_specs=pl.BlockSpec(memory_space=pl.ANY),
        out_shape=jax.ShapeDtypeStruct((C * _HW, D), jnp.float32),
        scratch_shapes=[pltpu.VMEM((_HW, _HW), jnp.bfloat16),
                        pltpu.VMEM((2, blk, D), jnp.float32),
                        pltpu.VMEM((2, blk, D), jnp.float32),
                        pltpu.SemaphoreType.DMA((2, 2)),
                        pltpu.SemaphoreType.DMA((2, 2))],
        compiler_params=pltpu.CompilerParams(
            dimension_semantics=("arbitrary",),
            vmem_limit_bytes=52 * 1024 * 1024),
    )(s, lmat, x2)

    return jnp.transpose(y2.reshape(C, _H, _W, D), (0, 3, 1, 2))
